# Initial kernel scaffold; baseline (speedup 1.0000x reference)
import jax
import jax.numpy as jnp
from jax.experimental import pallas as pl


def kernel(x, eps, W1, b1, g1, be1, W2, b2, g2, be2, Wl, bl, edge_index, batch, paper_count):
    raise NotImplementedError("write your pallas kernel here")


# speedup vs baseline: 26.5344x; 26.5344x over previous
import functools

import jax
import jax.numpy as jnp
from jax import lax
from jax.experimental import pallas as pl
from jax.experimental.pallas import tpu as pltpu
from jax.experimental.pallas import tpu_sc as plsc

N = 100000
H = 64
G = 1024
BN_EPS = 1e-5

NC = 2
NS = 16
NW = NC * NS
EB = 1024
OUTER = 196
E_PAD = NW * OUTER * EB
XROWS = 100096
RPT = XROWS // NS
DW = 8

BLK = 4000
NB = N // BLK

_HI = lax.Precision.HIGHEST


def _seg_mesh_kernel(xp, srcp, dstp, zeros, out, src_v, dst_v, rows, acc, sem):
    cid = lax.axis_index("c")
    sid = lax.axis_index("s")
    wid = sid * NC + cid

    pltpu.sync_copy(zeros.at[pl.ds(sid * RPT, RPT)], acc.at[pl.ds(sid * RPT, RPT)])
    plsc.subcore_barrier()

    def chunk(ci, carry):
        base = (wid * OUTER + ci) * EB
        pltpu.sync_copy(srcp.at[pl.ds(base, EB)], src_v)
        pltpu.sync_copy(dstp.at[pl.ds(base, EB)], dst_v)
        pltpu.async_copy(xp.at[src_v], rows, sem).wait()
        pltpu.sync_copy(rows, acc.at[dst_v], add=True)
        return carry

    lax.fori_loop(0, OUTER, chunk, jnp.int32(0))

    plsc.subcore_barrier()
    pltpu.sync_copy(
        acc.at[pl.ds(sid * RPT, RPT)],
        out.at[pl.ds(cid * XROWS + sid * RPT, RPT)],
    )


@functools.cache
def _seg_call():
    return functools.partial(
        pl.kernel,
        mesh=plsc.VectorSubcoreMesh(core_axis_name="c", subcore_axis_name="s"),
        compiler_params=pltpu.CompilerParams(use_tc_tiling_on_sc=False),
        out_type=jax.ShapeDtypeStruct((NC * XROWS, DW), jnp.float32),
        scratch_types=[
            pltpu.VMEM((EB,), jnp.int32),
            pltpu.VMEM((EB,), jnp.int32),
            pltpu.VMEM((EB, DW), jnp.float32),
            pltpu.VMEM_SHARED((XROWS, DW), jnp.float32),
            pltpu.SemaphoreType.DMA,
        ],
    )(_seg_mesh_kernel)


def _segment_partials(xp, srcp, dstp, zeros):
    return _seg_call()(xp, srcp, dstp, zeros)


def _t1_body(scale_ref, x_ref, p0_ref, p1_ref, w1_ref, b1_ref, h_ref, st_ref):
    i = pl.program_id(0)
    h = x_ref[...] * scale_ref[0, 0] + p0_ref[...] + p1_ref[...]
    h_ref[...] = h
    t1 = b1_ref[...]
    for a in range(4):
        t1 = t1 + h[:, a : a + 1] * w1_ref[a : a + 1, :]

    @pl.when(i == 0)
    def _():
        st_ref[...] = jnp.zeros_like(st_ref)

    st_ref[0:1, :] += jnp.sum(t1, axis=0, keepdims=True)
    st_ref[1:2, :] += jnp.sum(t1 * t1, axis=0, keepdims=True)


def _bn_coeffs(st_ref, g, be):
    mu = st_ref[0:1, :] * (1.0 / N)
    var = st_ref[1:2, :] * (1.0 / N) - mu * mu
    s = g * lax.rsqrt(var + BN_EPS)
    c = be - mu * s
    return s, c


def _mlp_t1(h, w1_ref, b1_ref):
    t1 = b1_ref[...]
    for a in range(4):
        t1 = t1 + h[:, a : a + 1] * w1_ref[a : a + 1, :]
    return t1


def _t2_body(h_ref, st1_ref, w1_ref, b1_ref, g1_ref, be1_ref, w2_ref, b2_ref,
             st2_ref):
    i = pl.program_id(0)
    s1, c1 = _bn_coeffs(st1_ref, g1_ref[...], be1_ref[...])
    h2 = jnp.maximum(_mlp_t1(h_ref[...], w1_ref, b1_ref) * s1 + c1, 0.0)
    t2 = lax.dot_general(h2, w2_ref[...], (((1,), (0,)), ((), ())),
                         precision=_HI, preferred_element_type=jnp.float32)
    t2 = t2 + b2_ref[...]

    @pl.when(i == 0)
    def _():
        st2_ref[...] = jnp.zeros_like(st2_ref)

    st2_ref[0:1, :] += jnp.sum(t2, axis=0, keepdims=True)
    st2_ref[1:2, :] += jnp.sum(t2 * t2, axis=0, keepdims=True)


def _t3_body(h_ref, b_ref, st1_ref, st2_ref, w1_ref, b1_ref, g1_ref, be1_ref,
             w2_ref, b2_ref, g2_ref, be2_ref, wl_ref, bl_ref, out_ref,
             pool_ref, cnt_ref):
    i = pl.program_id(0)
    s1, c1 = _bn_coeffs(st1_ref, g1_ref[...], be1_ref[...])
    s2, c2 = _bn_coeffs(st2_ref, g2_ref[...], be2_ref[...])
    h2 = jnp.maximum(_mlp_t1(h_ref[...], w1_ref, b1_ref) * s1 + c1, 0.0)
    t2 = lax.dot_general(h2, w2_ref[...], (((1,), (0,)), ((), ())),
                         precision=_HI, preferred_element_type=jnp.float32)
    h3 = jnp.maximum((t2 + b2_ref[...]) * s2 + c2, 0.0)

    b_row = b_ref[0]
    iota = lax.broadcasted_iota(jnp.int32, (G, BLK), 0)
    pt = (iota == b_row).astype(jnp.float32)

    @pl.when(i == 0)
    def _():
        pool_ref[...] = jnp.zeros_like(pool_ref)
        cnt_ref[...] = jnp.zeros_like(cnt_ref)

    pool_ref[...] += lax.dot_general(pt, h3, (((1,), (0,)), ((), ())),
                                     precision=_HI,
                                     preferred_element_type=jnp.float32)
    cnt_ref[...] += jnp.sum(pt, axis=1, keepdims=True)

    @pl.when(i == NB - 1)
    def _():
        mean = pool_ref[...] / jnp.maximum(cnt_ref[...], 1.0)
        logits = lax.dot_general(mean, wl_ref[...], (((1,), (0,)), ((), ())),
                                 precision=_HI,
                                 preferred_element_type=jnp.float32)
        logits = logits + bl_ref[...]
        m = jnp.max(logits, axis=1, keepdims=True)
        lse = m + jnp.log(jnp.sum(jnp.exp(logits - m), axis=1, keepdims=True))
        out_ref[...] = logits - lse


def _full_spec(shape):
    return pl.BlockSpec(shape, lambda i: tuple(0 for _ in shape))


def kernel(x, eps, W1, b1, g1, be1, W2, b2, g2, be2, Wl, bl, edge_index, batch,
           paper_count):
    f32 = jnp.float32
    pad_e = E_PAD - edge_index.shape[1]
    spread = N + (jnp.arange(pad_e, dtype=jnp.int32) % (XROWS - N))
    srcp = jnp.concatenate([edge_index[0], spread])
    dstp = jnp.concatenate([edge_index[1], spread])
    xp = jnp.pad(x, ((0, XROWS - N), (0, DW - 4)))
    zeros = jnp.zeros((XROWS, DW), f32)

    partials = _segment_partials(xp, srcp, dstp, zeros)
    p0 = partials[:N, :4]
    p1 = partials[XROWS : XROWS + N, :4]

    scale = jnp.reshape(1.0 + eps, (1, 1)).astype(f32)
    b1r = b1.reshape(1, 2 * H)
    g1r = g1.reshape(1, 2 * H)
    be1r = be1.reshape(1, 2 * H)
    b2r = b2.reshape(1, H)
    g2r = g2.reshape(1, H)
    be2r = be2.reshape(1, H)
    blr = bl.reshape(1, 2)

    row_spec = pl.BlockSpec((BLK, 4), lambda i: (i, 0))
    h, st1 = pl.pallas_call(
        _t1_body,
        grid=(NB,),
        in_specs=[
            pl.BlockSpec(memory_space=pltpu.SMEM),
            row_spec, row_spec, row_spec,
            _full_spec((4, 2 * H)),
            _full_spec((1, 2 * H)),
        ],
        out_specs=[
            pl.BlockSpec((BLK, 4), lambda i: (i, 0)),
            _full_spec((2, 2 * H)),
        ],
        out_shape=[
            jax.ShapeDtypeStruct((N, 4), f32),
            jax.ShapeDtypeStruct((2, 2 * H), f32),
        ],
    )(scale, x, p0, p1, W1, b1r)

    st2 = pl.pallas_call(
        _t2_body,
        grid=(NB,),
        in_specs=[
            row_spec,
            _full_spec((2, 2 * H)),
            _full_spec((4, 2 * H)),
            _full_spec((1, 2 * H)),
            _full_spec((1, 2 * H)),
            _full_spec((1, 2 * H)),
            _full_spec((2 * H, H)),
            _full_spec((1, H)),
        ],
        out_specs=_full_spec((2, H)),
        out_shape=jax.ShapeDtypeStruct((2, H), f32),
    )(h, st1, W1, b1r, g1r, be1r, W2, b2r)

    batch3 = batch.reshape(NB, 1, BLK)
    out = pl.pallas_call(
        _t3_body,
        grid=(NB,),
        in_specs=[
            row_spec,
            pl.BlockSpec((1, 1, BLK), lambda i: (i, 0, 0)),
            _full_spec((2, 2 * H)),
            _full_spec((2, H)),
            _full_spec((4, 2 * H)),
            _full_spec((1, 2 * H)),
            _full_spec((1, 2 * H)),
            _full_spec((1, 2 * H)),
            _full_spec((2 * H, H)),
            _full_spec((1, H)),
            _full_spec((1, H)),
            _full_spec((1, H)),
            _full_spec((H, 2)),
            _full_spec((1, 2)),
        ],
        out_specs=_full_spec((G, 2)),
        out_shape=jax.ShapeDtypeStruct((G, 2), f32),
        scratch_shapes=[
            pltpu.VMEM((G, H), f32),
            pltpu.VMEM((G, 1), f32),
        ],
    )(h, batch3, st1, st2, W1, b1r, g1r, be1r, W2, b2r, g2r, be2r, Wl, blr)
    return out


# speedup vs baseline: 34.4289x; 1.2975x over previous
import functools

import jax
import jax.numpy as jnp
from jax import lax
from jax.experimental import pallas as pl
from jax.experimental.pallas import tpu as pltpu
from jax.experimental.pallas import tpu_sc as plsc

N = 100000
H = 64
G = 1024
BN_EPS = 1e-5

NC = 2
NS = 16
NW = NC * NS
EB = 1024
EW = 6400000 // NW
OUTER = EW // EB
EB_TAIL = EW - OUTER * EB
XROWS = 104000
RPT = XROWS // NS
DW = 8

BLK = 4000
NB = N // BLK

_HI = lax.Precision.HIGHEST


def _seg_mesh_kernel(xp, edges, zeros, out, src_v, dst_v, rows,
                     src_t, dst_t, rows_t, acc, sem):
    cid = lax.axis_index("c")
    sid = lax.axis_index("s")
    wid = sid * NC + cid

    pltpu.sync_copy(zeros.at[pl.ds(sid * RPT, RPT)], acc.at[pl.ds(sid * RPT, RPT)])
    plsc.subcore_barrier()

    def chunk(ci, carry):
        base = wid * EW + ci * EB
        pltpu.sync_copy(edges.at[0, pl.ds(base, EB)], src_v)
        pltpu.sync_copy(edges.at[1, pl.ds(base, EB)], dst_v)
        pltpu.async_copy(xp.at[src_v], rows, sem).wait()
        pltpu.sync_copy(rows, acc.at[dst_v], add=True)
        return carry

    lax.fori_loop(0, OUTER, chunk, jnp.int32(0))

    base = wid * EW + OUTER * EB
    pltpu.sync_copy(edges.at[0, pl.ds(base, EB_TAIL)], src_t)
    pltpu.sync_copy(edges.at[1, pl.ds(base, EB_TAIL)], dst_t)
    pltpu.async_copy(xp.at[src_t], rows_t, sem).wait()
    pltpu.sync_copy(rows_t, acc.at[dst_t], add=True)

    plsc.subcore_barrier()
    pltpu.sync_copy(
        acc.at[pl.ds(sid * RPT, RPT)],
        out.at[pl.ds(cid * XROWS + sid * RPT, RPT)],
    )


@functools.cache
def _seg_call():
    return functools.partial(
        pl.kernel,
        mesh=plsc.VectorSubcoreMesh(core_axis_name="c", subcore_axis_name="s"),
        compiler_params=pltpu.CompilerParams(use_tc_tiling_on_sc=False),
        out_type=jax.ShapeDtypeStruct((NC * XROWS, DW), jnp.float32),
        scratch_types=[
            pltpu.VMEM((EB,), jnp.int32),
            pltpu.VMEM((EB,), jnp.int32),
            pltpu.VMEM((EB, DW), jnp.float32),
            pltpu.VMEM((EB_TAIL,), jnp.int32),
            pltpu.VMEM((EB_TAIL,), jnp.int32),
            pltpu.VMEM((EB_TAIL, DW), jnp.float32),
            pltpu.VMEM_SHARED((XROWS, DW), jnp.float32),
            pltpu.SemaphoreType.DMA,
        ],
    )(_seg_mesh_kernel)


def _segment_partials(xp, edges, zeros):
    return _seg_call()(xp, edges, zeros)


def _t1_body(scale_ref, x_ref, p0_ref, p1_ref, w1_ref, b1_ref, h_ref, st_ref):
    i = pl.program_id(0)
    h = x_ref[...] * scale_ref[0, 0] + p0_ref[:, :4] + p1_ref[:, :4]
    h_ref[...] = h
    t1 = b1_ref[...]
    for a in range(4):
        t1 = t1 + h[:, a : a + 1] * w1_ref[a : a + 1, :]

    @pl.when(i == 0)
    def _():
        st_ref[...] = jnp.zeros_like(st_ref)

    st_ref[0:1, :] += jnp.sum(t1, axis=0, keepdims=True)
    st_ref[1:2, :] += jnp.sum(t1 * t1, axis=0, keepdims=True)


def _bn_coeffs(st_ref, g, be):
    mu = st_ref[0:1, :] * (1.0 / N)
    var = st_ref[1:2, :] * (1.0 / N) - mu * mu
    s = g * lax.rsqrt(var + BN_EPS)
    c = be - mu * s
    return s, c


def _mlp_t1(h, w1_ref, b1_ref):
    t1 = b1_ref[...]
    for a in range(4):
        t1 = t1 + h[:, a : a + 1] * w1_ref[a : a + 1, :]
    return t1


def _t2_body(h_ref, st1_ref, w1_ref, b1_ref, g1_ref, be1_ref, w2_ref, b2_ref,
             st2_ref):
    i = pl.program_id(0)
    s1, c1 = _bn_coeffs(st1_ref, g1_ref[...], be1_ref[...])
    h2 = jnp.maximum(_mlp_t1(h_ref[...], w1_ref, b1_ref) * s1 + c1, 0.0)
    t2 = lax.dot_general(h2, w2_ref[...], (((1,), (0,)), ((), ())),
                         precision=_HI, preferred_element_type=jnp.float32)
    t2 = t2 + b2_ref[...]

    @pl.when(i == 0)
    def _():
        st2_ref[...] = jnp.zeros_like(st2_ref)

    st2_ref[0:1, :] += jnp.sum(t2, axis=0, keepdims=True)
    st2_ref[1:2, :] += jnp.sum(t2 * t2, axis=0, keepdims=True)


def _t3_body(h_ref, b_ref, st1_ref, st2_ref, w1_ref, b1_ref, g1_ref, be1_ref,
             w2_ref, b2_ref, g2_ref, be2_ref, wl_ref, bl_ref, out_ref,
             pool_ref, cnt_ref):
    i = pl.program_id(0)
    s1, c1 = _bn_coeffs(st1_ref, g1_ref[...], be1_ref[...])
    s2, c2 = _bn_coeffs(st2_ref, g2_ref[...], be2_ref[...])
    h2 = jnp.maximum(_mlp_t1(h_ref[...], w1_ref, b1_ref) * s1 + c1, 0.0)
    t2 = lax.dot_general(h2, w2_ref[...], (((1,), (0,)), ((), ())),
                         precision=_HI, preferred_element_type=jnp.float32)
    h3 = jnp.maximum((t2 + b2_ref[...]) * s2 + c2, 0.0)

    b_row = b_ref[0]
    iota = lax.broadcasted_iota(jnp.int32, (G, BLK), 0)
    m = iota == b_row
    pt = m.astype(jnp.bfloat16)

    @pl.when(i == 0)
    def _():
        pool_ref[...] = jnp.zeros_like(pool_ref)
        cnt_ref[...] = jnp.zeros_like(cnt_ref)

    pool_ref[...] += lax.dot_general(pt, h3.astype(jnp.bfloat16),
                                     (((1,), (0,)), ((), ())),
                                     preferred_element_type=jnp.float32)
    cnt_ref[...] += jnp.sum(jnp.where(m, 1.0, 0.0), axis=1, keepdims=True)

    @pl.when(i == NB - 1)
    def _():
        mean = pool_ref[...] / jnp.maximum(cnt_ref[...], 1.0)
        logits = lax.dot_general(mean, wl_ref[...], (((1,), (0,)), ((), ())),
                                 precision=_HI,
                                 preferred_element_type=jnp.float32)
        logits = logits + bl_ref[...]
        m = jnp.max(logits, axis=1, keepdims=True)
        lse = m + jnp.log(jnp.sum(jnp.exp(logits - m), axis=1, keepdims=True))
        out_ref[...] = logits - lse


def _full_spec(shape):
    return pl.BlockSpec(shape, lambda i: tuple(0 for _ in shape))


def kernel(x, eps, W1, b1, g1, be1, W2, b2, g2, be2, Wl, bl, edge_index, batch,
           paper_count):
    f32 = jnp.float32
    xp = jnp.pad(x, ((0, XROWS - N), (0, DW - 4)))
    zeros = jnp.zeros((XROWS, DW), f32)

    partials = _segment_partials(xp, edge_index, zeros)

    scale = jnp.reshape(1.0 + eps, (1, 1)).astype(f32)
    b1r = b1.reshape(1, 2 * H)
    g1r = g1.reshape(1, 2 * H)
    be1r = be1.reshape(1, 2 * H)
    b2r = b2.reshape(1, H)
    g2r = g2.reshape(1, H)
    be2r = be2.reshape(1, H)
    blr = bl.reshape(1, 2)

    row_spec = pl.BlockSpec((BLK, 4), lambda i: (i, 0))
    pblks = XROWS // BLK
    h, st1 = pl.pallas_call(
        _t1_body,
        grid=(NB,),
        in_specs=[
            pl.BlockSpec(memory_space=pltpu.SMEM),
            row_spec,
            pl.BlockSpec((BLK, DW), lambda i: (i, 0)),
            pl.BlockSpec((BLK, DW), lambda i: (i + pblks, 0)),
            _full_spec((4, 2 * H)),
            _full_spec((1, 2 * H)),
        ],
        out_specs=[
            pl.BlockSpec((BLK, 4), lambda i: (i, 0)),
            _full_spec((2, 2 * H)),
        ],
        out_shape=[
            jax.ShapeDtypeStruct((N, 4), f32),
            jax.ShapeDtypeStruct((2, 2 * H), f32),
        ],
    )(scale, x, partials, partials, W1, b1r)

    st2 = pl.pallas_call(
        _t2_body,
        grid=(NB,),
        in_specs=[
            row_spec,
            _full_spec((2, 2 * H)),
            _full_spec((4, 2 * H)),
            _full_spec((1, 2 * H)),
            _full_spec((1, 2 * H)),
            _full_spec((1, 2 * H)),
            _full_spec((2 * H, H)),
            _full_spec((1, H)),
        ],
        out_specs=_full_spec((2, H)),
        out_shape=jax.ShapeDtypeStruct((2, H), f32),
    )(h, st1, W1, b1r, g1r, be1r, W2, b2r)

    batch3 = batch.reshape(NB, 1, BLK)
    out = pl.pallas_call(
        _t3_body,
        grid=(NB,),
        in_specs=[
            row_spec,
            pl.BlockSpec((1, 1, BLK), lambda i: (i, 0, 0)),
            _full_spec((2, 2 * H)),
            _full_spec((2, H)),
            _full_spec((4, 2 * H)),
            _full_spec((1, 2 * H)),
            _full_spec((1, 2 * H)),
            _full_spec((1, 2 * H)),
            _full_spec((2 * H, H)),
            _full_spec((1, H)),
            _full_spec((1, H)),
            _full_spec((1, H)),
            _full_spec((H, 2)),
            _full_spec((1, 2)),
        ],
        out_specs=_full_spec((G, 2)),
        out_shape=jax.ShapeDtypeStruct((G, 2), f32),
        scratch_shapes=[
            pltpu.VMEM((G, H), f32),
            pltpu.VMEM((G, 1), f32),
        ],
    )(h, batch3, st1, st2, W1, b1r, g1r, be1r, W2, b2r, g2r, be2r, Wl, blr)
    return out


# speedup vs baseline: 45.9117x; 1.3335x over previous
import functools

import jax
import jax.numpy as jnp
from jax import lax
from jax.experimental import pallas as pl
from jax.experimental.pallas import tpu as pltpu
from jax.experimental.pallas import tpu_sc as plsc

N = 100000
H = 64
G = 1024
BN_EPS = 1e-5

NC = 2
NS = 16
NW = NC * NS
EB = 1000
EW = 6400000 // NW
NBUF = 4
ITERS = EW // (EB * NBUF)
XROWS = 104000
RPT = XROWS // NS
DW = 8

BLK = 4000
NB = N // BLK

_HI = lax.Precision.HIGHEST


def _seg_mesh_kernel(xp, edges, zeros, out,
                     src_v, dst_v, rows, acc, semi, semg, sems):
    cid = lax.axis_index("c")
    sid = lax.axis_index("s")
    wid = sid * NC + cid

    pltpu.sync_copy(zeros.at[pl.ds(sid * RPT, RPT)], acc.at[pl.ds(sid * RPT, RPT)])
    plsc.subcore_barrier()

    def drain_scatters():
        for b in range(NBUF):
            pltpu.make_async_copy(xp.at[pl.ds(0, EB)], rows[b], sems).wait()

    def body(k, carry):
        base = wid * EW + k * (NBUF * EB)
        idx_d = []
        for b in range(NBUF):
            idx_d.append(pltpu.async_copy(
                edges.at[0, pl.ds(base + b * EB, EB)], src_v[b], semi))
            idx_d.append(pltpu.async_copy(
                edges.at[1, pl.ds(base + b * EB, EB)], dst_v[b], semi))
        pl.when(k > 0)(drain_scatters)
        for d in idx_d:
            d.wait()
        gat_d = [pltpu.async_copy(xp.at[src_v[b]], rows[b], semg)
                 for b in range(NBUF)]
        for b in range(NBUF):
            gat_d[b].wait()
            pltpu.async_copy(rows[b], acc.at[dst_v[b]], sems, add=True)
        return carry

    lax.fori_loop(0, ITERS, body, jnp.int32(0))
    drain_scatters()

    plsc.subcore_barrier()
    pltpu.sync_copy(
        acc.at[pl.ds(sid * RPT, RPT)],
        out.at[pl.ds(cid * XROWS + sid * RPT, RPT)],
    )


@functools.cache
def _seg_call():
    return functools.partial(
        pl.kernel,
        mesh=plsc.VectorSubcoreMesh(core_axis_name="c", subcore_axis_name="s"),
        compiler_params=pltpu.CompilerParams(use_tc_tiling_on_sc=False),
        out_type=jax.ShapeDtypeStruct((NC * XROWS, DW), jnp.float32),
        scratch_types=[
            [pltpu.VMEM((EB,), jnp.int32) for _ in range(NBUF)],
            [pltpu.VMEM((EB,), jnp.int32) for _ in range(NBUF)],
            [pltpu.VMEM((EB, DW), jnp.float32) for _ in range(NBUF)],
            pltpu.VMEM_SHARED((XROWS, DW), jnp.float32),
            pltpu.SemaphoreType.DMA,
            pltpu.SemaphoreType.DMA,
            pltpu.SemaphoreType.DMA,
        ],
    )(_seg_mesh_kernel)


def _segment_partials(xp, edges, zeros):
    return _seg_call()(xp, edges, zeros)


def _t1_body(scale_ref, x_ref, p0_ref, p1_ref, w1_ref, b1_ref, h_ref, st_ref):
    i = pl.program_id(0)
    h = x_ref[...] * scale_ref[0, 0] + p0_ref[:, :4] + p1_ref[:, :4]
    h_ref[...] = h
    t1 = b1_ref[...]
    for a in range(4):
        t1 = t1 + h[:, a : a + 1] * w1_ref[a : a + 1, :]

    @pl.when(i == 0)
    def _():
        st_ref[...] = jnp.zeros_like(st_ref)

    st_ref[0:1, :] += jnp.sum(t1, axis=0, keepdims=True)
    st_ref[1:2, :] += jnp.sum(t1 * t1, axis=0, keepdims=True)


def _bn_coeffs(st_ref, g, be):
    mu = st_ref[0:1, :] * (1.0 / N)
    var = st_ref[1:2, :] * (1.0 / N) - mu * mu
    s = g * lax.rsqrt(var + BN_EPS)
    c = be - mu * s
    return s, c


def _mlp_t1(h, w1_ref, b1_ref):
    t1 = b1_ref[...]
    for a in range(4):
        t1 = t1 + h[:, a : a + 1] * w1_ref[a : a + 1, :]
    return t1


def _t2_body(h_ref, st1_ref, w1_ref, b1_ref, g1_ref, be1_ref, w2_ref, b2_ref,
             st2_ref):
    i = pl.program_id(0)
    s1, c1 = _bn_coeffs(st1_ref, g1_ref[...], be1_ref[...])
    h2 = jnp.maximum(_mlp_t1(h_ref[...], w1_ref, b1_ref) * s1 + c1, 0.0)
    t2 = lax.dot_general(h2, w2_ref[...], (((1,), (0,)), ((), ())),
                         precision=_HI, preferred_element_type=jnp.float32)
    t2 = t2 + b2_ref[...]

    @pl.when(i == 0)
    def _():
        st2_ref[...] = jnp.zeros_like(st2_ref)

    st2_ref[0:1, :] += jnp.sum(t2, axis=0, keepdims=True)
    st2_ref[1:2, :] += jnp.sum(t2 * t2, axis=0, keepdims=True)


def _t3_body(h_ref, b_ref, st1_ref, st2_ref, w1_ref, b1_ref, g1_ref, be1_ref,
             w2_ref, b2_ref, g2_ref, be2_ref, wl_ref, bl_ref, out_ref,
             pool_ref, cnt_ref):
    i = pl.program_id(0)
    s1, c1 = _bn_coeffs(st1_ref, g1_ref[...], be1_ref[...])
    s2, c2 = _bn_coeffs(st2_ref, g2_ref[...], be2_ref[...])
    h2 = jnp.maximum(_mlp_t1(h_ref[...], w1_ref, b1_ref) * s1 + c1, 0.0)
    t2 = lax.dot_general(h2, w2_ref[...], (((1,), (0,)), ((), ())),
                         precision=_HI, preferred_element_type=jnp.float32)
    h3 = jnp.maximum((t2 + b2_ref[...]) * s2 + c2, 0.0)

    b_row = b_ref[0]
    iota = lax.broadcasted_iota(jnp.int32, (G, BLK), 0)
    m = iota == b_row
    pt = m.astype(jnp.bfloat16)

    @pl.when(i == 0)
    def _():
        pool_ref[...] = jnp.zeros_like(pool_ref)
        cnt_ref[...] = jnp.zeros_like(cnt_ref)

    pool_ref[...] += lax.dot_general(pt, h3.astype(jnp.bfloat16),
                                     (((1,), (0,)), ((), ())),
                                     preferred_element_type=jnp.float32)
    cnt_ref[...] += jnp.sum(jnp.where(m, 1.0, 0.0), axis=1, keepdims=True)

    @pl.when(i == NB - 1)
    def _():
        mean = pool_ref[...] / jnp.maximum(cnt_ref[...], 1.0)
        logits = lax.dot_general(mean, wl_ref[...], (((1,), (0,)), ((), ())),
                                 precision=_HI,
                                 preferred_element_type=jnp.float32)
        logits = logits + bl_ref[...]
        m = jnp.max(logits, axis=1, keepdims=True)
        lse = m + jnp.log(jnp.sum(jnp.exp(logits - m), axis=1, keepdims=True))
        out_ref[...] = logits - lse


def _full_spec(shape):
    return pl.BlockSpec(shape, lambda i: tuple(0 for _ in shape))


def kernel(x, eps, W1, b1, g1, be1, W2, b2, g2, be2, Wl, bl, edge_index, batch,
           paper_count):
    f32 = jnp.float32
    xp = jnp.pad(x, ((0, XROWS - N), (0, DW - 4)))
    zeros = jnp.zeros((XROWS, DW), f32)

    partials = _segment_partials(xp, edge_index, zeros)

    scale = jnp.reshape(1.0 + eps, (1, 1)).astype(f32)
    b1r = b1.reshape(1, 2 * H)
    g1r = g1.reshape(1, 2 * H)
    be1r = be1.reshape(1, 2 * H)
    b2r = b2.reshape(1, H)
    g2r = g2.reshape(1, H)
    be2r = be2.reshape(1, H)
    blr = bl.reshape(1, 2)

    row_spec = pl.BlockSpec((BLK, 4), lambda i: (i, 0))
    pblks = XROWS // BLK
    h, st1 = pl.pallas_call(
        _t1_body,
        grid=(NB,),
        in_specs=[
            pl.BlockSpec(memory_space=pltpu.SMEM),
            row_spec,
            pl.BlockSpec((BLK, DW), lambda i: (i, 0)),
            pl.BlockSpec((BLK, DW), lambda i: (i + pblks, 0)),
            _full_spec((4, 2 * H)),
            _full_spec((1, 2 * H)),
        ],
        out_specs=[
            pl.BlockSpec((BLK, 4), lambda i: (i, 0)),
            _full_spec((2, 2 * H)),
        ],
        out_shape=[
            jax.ShapeDtypeStruct((N, 4), f32),
            jax.ShapeDtypeStruct((2, 2 * H), f32),
        ],
    )(scale, x, partials, partials, W1, b1r)

    st2 = pl.pallas_call(
        _t2_body,
        grid=(NB,),
        in_specs=[
            row_spec,
            _full_spec((2, 2 * H)),
            _full_spec((4, 2 * H)),
            _full_spec((1, 2 * H)),
            _full_spec((1, 2 * H)),
            _full_spec((1, 2 * H)),
            _full_spec((2 * H, H)),
            _full_spec((1, H)),
        ],
        out_specs=_full_spec((2, H)),
        out_shape=jax.ShapeDtypeStruct((2, H), f32),
    )(h, st1, W1, b1r, g1r, be1r, W2, b2r)

    batch3 = batch.reshape(NB, 1, BLK)
    out = pl.pallas_call(
        _t3_body,
        grid=(NB,),
        in_specs=[
            row_spec,
            pl.BlockSpec((1, 1, BLK), lambda i: (i, 0, 0)),
            _full_spec((2, 2 * H)),
            _full_spec((2, H)),
            _full_spec((4, 2 * H)),
            _full_spec((1, 2 * H)),
            _full_spec((1, 2 * H)),
            _full_spec((1, 2 * H)),
            _full_spec((2 * H, H)),
            _full_spec((1, H)),
            _full_spec((1, H)),
            _full_spec((1, H)),
            _full_spec((H, 2)),
            _full_spec((1, 2)),
        ],
        out_specs=_full_spec((G, 2)),
        out_shape=jax.ShapeDtypeStruct((G, 2), f32),
        scratch_shapes=[
            pltpu.VMEM((G, H), f32),
            pltpu.VMEM((G, 1), f32),
        ],
    )(h, batch3, st1, st2, W1, b1r, g1r, be1r, W2, b2r, g2r, be2r, Wl, blr)
    return out


# speedup vs baseline: 47.5741x; 1.0362x over previous
import functools

import jax
import jax.numpy as jnp
from jax import lax
from jax.experimental import pallas as pl
from jax.experimental.pallas import tpu as pltpu
from jax.experimental.pallas import tpu_sc as plsc

N = 100000
H = 64
G = 1024
BN_EPS = 1e-5

NC = 2
NS = 16
NW = NC * NS
EB = 1000
EW = 6400000 // NW
NBUF = 4
ITERS = EW // (EB * NBUF)
XROWS = 104000
RPT = XROWS // NS
DW = 8

BLK = 4000
NB = N // BLK

_HI = lax.Precision.HIGHEST


def _seg_mesh_kernel(xp, edges, zeros, out,
                     src_v, dst_v, rows, acc, semi, semg, sems):
    cid = lax.axis_index("c")
    sid = lax.axis_index("s")
    wid = sid * NC + cid

    pltpu.sync_copy(zeros.at[pl.ds(sid * RPT, RPT)], acc.at[pl.ds(sid * RPT, RPT)])
    plsc.subcore_barrier()

    def drain_scatters():
        for b in range(NBUF):
            pltpu.make_async_copy(xp.at[pl.ds(0, EB)], rows[b], sems).wait()

    def body(k, carry):
        base = wid * EW + k * (NBUF * EB)
        idx_d = []
        for b in range(NBUF):
            idx_d.append(pltpu.async_copy(
                edges.at[0, pl.ds(base + b * EB, EB)], src_v[b], semi))
            idx_d.append(pltpu.async_copy(
                edges.at[1, pl.ds(base + b * EB, EB)], dst_v[b], semi))
        pl.when(k > 0)(drain_scatters)
        for d in idx_d:
            d.wait()
        gat_d = [pltpu.async_copy(xp.at[src_v[b]], rows[b], semg)
                 for b in range(NBUF)]
        for b in range(NBUF):
            gat_d[b].wait()
            pltpu.async_copy(rows[b], acc.at[dst_v[b]], sems, add=True)
        return carry

    lax.fori_loop(0, ITERS, body, jnp.int32(0))
    drain_scatters()

    plsc.subcore_barrier()
    pltpu.sync_copy(
        acc.at[pl.ds(sid * RPT, RPT)],
        out.at[pl.ds(cid * XROWS + sid * RPT, RPT)],
    )


@functools.cache
def _seg_call():
    return functools.partial(
        pl.kernel,
        mesh=plsc.VectorSubcoreMesh(core_axis_name="c", subcore_axis_name="s"),
        compiler_params=pltpu.CompilerParams(use_tc_tiling_on_sc=False),
        out_type=jax.ShapeDtypeStruct((NC * XROWS, DW), jnp.float32),
        scratch_types=[
            [pltpu.VMEM((EB,), jnp.int32) for _ in range(NBUF)],
            [pltpu.VMEM((EB,), jnp.int32) for _ in range(NBUF)],
            [pltpu.VMEM((EB, DW), jnp.float32) for _ in range(NBUF)],
            pltpu.VMEM_SHARED((XROWS, DW), jnp.float32),
            pltpu.SemaphoreType.DMA,
            pltpu.SemaphoreType.DMA,
            pltpu.SemaphoreType.DMA,
        ],
    )(_seg_mesh_kernel)


def _segment_partials(xp, edges, zeros):
    return _seg_call()(xp, edges, zeros)


def _t1_body(scale_ref, x_ref, p0_ref, p1_ref, w1_ref, b1_ref, h_ref, st_ref):
    i = pl.program_id(0)
    h = x_ref[...] * scale_ref[0, 0] + p0_ref[:, :4] + p1_ref[:, :4]
    h_ref[...] = h
    t1 = b1_ref[...]
    for a in range(4):
        t1 = t1 + h[:, a : a + 1] * w1_ref[a : a + 1, :]

    @pl.when(i == 0)
    def _():
        st_ref[...] = jnp.zeros_like(st_ref)

    st_ref[0:1, :] += jnp.sum(t1, axis=0, keepdims=True)
    st_ref[1:2, :] += jnp.sum(t1 * t1, axis=0, keepdims=True)


def _bn_coeffs(st_ref, g, be):
    mu = st_ref[0:1, :] * (1.0 / N)
    var = st_ref[1:2, :] * (1.0 / N) - mu * mu
    s = g * lax.rsqrt(var + BN_EPS)
    c = be - mu * s
    return s, c


def _mlp_t1(h, w1_ref, b1_ref):
    t1 = b1_ref[...]
    for a in range(4):
        t1 = t1 + h[:, a : a + 1] * w1_ref[a : a + 1, :]
    return t1


def _t2_body(h_ref, st1_ref, w1_ref, b1_ref, g1_ref, be1_ref, w2_ref, b2_ref,
             st2_ref):
    i = pl.program_id(0)
    s1, c1 = _bn_coeffs(st1_ref, g1_ref[...], be1_ref[...])
    h2 = jnp.maximum(_mlp_t1(h_ref[...], w1_ref, b1_ref) * s1 + c1, 0.0)
    t2 = lax.dot_general(h2, w2_ref[...], (((1,), (0,)), ((), ())),
                         precision=_HI, preferred_element_type=jnp.float32)
    t2 = t2 + b2_ref[...]

    @pl.when(i == 0)
    def _():
        st2_ref[...] = jnp.zeros_like(st2_ref)

    st2_ref[0:1, :] += jnp.sum(t2, axis=0, keepdims=True)
    st2_ref[1:2, :] += jnp.sum(t2 * t2, axis=0, keepdims=True)


def _t3_body(h_ref, b_ref, st1_ref, st2_ref, w1_ref, b1_ref, g1_ref, be1_ref,
             w2_ref, b2_ref, g2_ref, be2_ref, wl_ref, bl_ref, out_ref,
             pool_ref):
    i = pl.program_id(0)
    s1, c1 = _bn_coeffs(st1_ref, g1_ref[...], be1_ref[...])
    s2, c2 = _bn_coeffs(st2_ref, g2_ref[...], be2_ref[...])
    h2 = jnp.maximum(_mlp_t1(h_ref[...], w1_ref, b1_ref) * s1 + c1, 0.0)
    t2 = lax.dot_general(h2, w2_ref[...], (((1,), (0,)), ((), ())),
                         precision=_HI, preferred_element_type=jnp.float32)
    h3 = jnp.maximum((t2 + b2_ref[...]) * s2 + c2, 0.0)

    b_row = b_ref[0]
    iota = lax.broadcasted_iota(jnp.int32, (G, BLK), 0)
    pt = (iota == b_row).astype(jnp.bfloat16)
    aug = jnp.concatenate(
        [h3.astype(jnp.bfloat16), jnp.ones((BLK, 1), jnp.bfloat16)], axis=1)

    @pl.when(i == 0)
    def _():
        pool_ref[...] = jnp.zeros_like(pool_ref)

    pool_ref[...] += lax.dot_general(pt, aug, (((1,), (0,)), ((), ())),
                                     preferred_element_type=jnp.float32)

    @pl.when(i == NB - 1)
    def _():
        cnt = pool_ref[:, H : H + 1]
        mean = pool_ref[:, :H] / jnp.maximum(cnt, 1.0)
        logits = lax.dot_general(mean, wl_ref[...], (((1,), (0,)), ((), ())),
                                 precision=_HI,
                                 preferred_element_type=jnp.float32)
        logits = logits + bl_ref[...]
        m = jnp.max(logits, axis=1, keepdims=True)
        lse = m + jnp.log(jnp.sum(jnp.exp(logits - m), axis=1, keepdims=True))
        out_ref[...] = logits - lse


def _full_spec(shape):
    return pl.BlockSpec(shape, lambda i: tuple(0 for _ in shape))


def kernel(x, eps, W1, b1, g1, be1, W2, b2, g2, be2, Wl, bl, edge_index, batch,
           paper_count):
    f32 = jnp.float32
    xp = jnp.pad(x, ((0, XROWS - N), (0, DW - 4)))
    zeros = jnp.zeros((XROWS, DW), f32)

    partials = _segment_partials(xp, edge_index, zeros)

    scale = jnp.reshape(1.0 + eps, (1, 1)).astype(f32)
    b1r = b1.reshape(1, 2 * H)
    g1r = g1.reshape(1, 2 * H)
    be1r = be1.reshape(1, 2 * H)
    b2r = b2.reshape(1, H)
    g2r = g2.reshape(1, H)
    be2r = be2.reshape(1, H)
    blr = bl.reshape(1, 2)

    row_spec = pl.BlockSpec((BLK, 4), lambda i: (i, 0))
    pblks = XROWS // BLK
    h, st1 = pl.pallas_call(
        _t1_body,
        grid=(NB,),
        in_specs=[
            pl.BlockSpec(memory_space=pltpu.SMEM),
            row_spec,
            pl.BlockSpec((BLK, DW), lambda i: (i, 0)),
            pl.BlockSpec((BLK, DW), lambda i: (i + pblks, 0)),
            _full_spec((4, 2 * H)),
            _full_spec((1, 2 * H)),
        ],
        out_specs=[
            pl.BlockSpec((BLK, 4), lambda i: (i, 0)),
            _full_spec((2, 2 * H)),
        ],
        out_shape=[
            jax.ShapeDtypeStruct((N, 4), f32),
            jax.ShapeDtypeStruct((2, 2 * H), f32),
        ],
    )(scale, x, partials, partials, W1, b1r)

    st2 = pl.pallas_call(
        _t2_body,
        grid=(NB,),
        in_specs=[
            row_spec,
            _full_spec((2, 2 * H)),
            _full_spec((4, 2 * H)),
            _full_spec((1, 2 * H)),
            _full_spec((1, 2 * H)),
            _full_spec((1, 2 * H)),
            _full_spec((2 * H, H)),
            _full_spec((1, H)),
        ],
        out_specs=_full_spec((2, H)),
        out_shape=jax.ShapeDtypeStruct((2, H), f32),
    )(h, st1, W1, b1r, g1r, be1r, W2, b2r)

    batch3 = batch.reshape(NB, 1, BLK)
    out = pl.pallas_call(
        _t3_body,
        grid=(NB,),
        in_specs=[
            row_spec,
            pl.BlockSpec((1, 1, BLK), lambda i: (i, 0, 0)),
            _full_spec((2, 2 * H)),
            _full_spec((2, H)),
            _full_spec((4, 2 * H)),
            _full_spec((1, 2 * H)),
            _full_spec((1, 2 * H)),
            _full_spec((1, 2 * H)),
            _full_spec((2 * H, H)),
            _full_spec((1, H)),
            _full_spec((1, H)),
            _full_spec((1, H)),
            _full_spec((H, 2)),
            _full_spec((1, 2)),
        ],
        out_specs=_full_spec((G, 2)),
        out_shape=jax.ShapeDtypeStruct((G, 2), f32),
        scratch_shapes=[
            pltpu.VMEM((G, H + 1), f32),
        ],
    )(h, batch3, st1, st2, W1, b1r, g1r, be1r, W2, b2r, g2r, be2r, Wl, blr)
    return out
